# Initial kernel scaffold; baseline (speedup 1.0000x reference)
import jax
import jax.numpy as jnp
from jax.experimental import pallas as pl


def kernel(inputs, W_body, b_body, embed):
    raise NotImplementedError("write your pallas kernel here")


# speedup vs baseline: 1.4264x; 1.4264x over previous
import functools

import jax
import jax.numpy as jnp
from jax import lax
from jax.experimental import pallas as pl
from jax.experimental.pallas import tpu as pltpu
from jax.experimental.pallas import tpu_sc as plsc

N_TOK = 16384
IN_DIM = 768
EMBED_DIM = 256
N_EMBED = 8192

TM = 1024
MT = N_TOK // TM
NC = 1024
NCHUNK = N_EMBED // NC

_BIG = 3.0e38


def _argmin_body(x_ref, w_ref, b_ref, e_ref, idx_ref, esq_ref):
    @pl.when(pl.program_id(0) == 0)
    def _():
        e = e_ref[...]
        esq_ref[0, :] = jnp.sum(e * e, axis=1)

    xs = lax.dot_general(x_ref[...], w_ref[...], (((1,), (0,)), ((), ())),
                         preferred_element_type=jnp.float32)
    xs = xs + b_ref[...]
    xsq = jnp.sum(xs * xs, axis=1, keepdims=True)

    def chunk(c, carry):
        best_val, best_idx = carry
        e_c = e_ref[pl.ds(c * NC, NC), :]
        mm = lax.dot_general(xs, e_c, (((1,), (1,)), ((), ())),
                             preferred_element_type=jnp.float32)
        esq_c = esq_ref[:, pl.ds(c * NC, NC)]
        dist = (xsq + esq_c) - 2.0 * mm
        m = jnp.min(dist, axis=1, keepdims=True)
        lane = lax.broadcasted_iota(jnp.int32, (TM, NC), 1).astype(jnp.float32)
        cand = jnp.where(dist == m, lane, _BIG)
        a = jnp.min(cand, axis=1, keepdims=True) + jnp.float32(NC) * c
        upd = m < best_val
        return jnp.where(upd, m, best_val), jnp.where(upd, a, best_idx)

    init = (jnp.full((TM, 1), _BIG, jnp.float32),
            jnp.zeros((TM, 1), jnp.float32))
    _, best_idx = lax.fori_loop(0, NCHUNK, chunk, init)
    idx_ref[...] = best_idx.astype(jnp.int32).reshape(1, 1, TM)


@jax.jit
def _argmin_call(inputs, w, b2d, embed):
    return pl.pallas_call(
        _argmin_body,
        grid=(MT,),
        in_specs=[
            pl.BlockSpec((TM, IN_DIM), lambda i: (i, 0)),
            pl.BlockSpec((IN_DIM, EMBED_DIM), lambda i: (0, 0)),
            pl.BlockSpec((1, EMBED_DIM), lambda i: (0, 0)),
            pl.BlockSpec((N_EMBED, EMBED_DIM), lambda i: (0, 0)),
        ],
        out_specs=pl.BlockSpec((1, 1, TM), lambda i: (i, 0, 0)),
        out_shape=jax.ShapeDtypeStruct((MT, 1, TM), jnp.int32),
        scratch_shapes=[pltpu.VMEM((1, N_EMBED), jnp.float32)],
        compiler_params=pltpu.CompilerParams(
            dimension_semantics=("arbitrary",)),
    )(inputs, w, b2d, embed)


_NW = 32
_BPW = N_TOK // _NW
_GC = 256


def _gather_body(table_hbm, idx_hbm, out_hbm, idx_v, rows_v, sem):
    wid = lax.axis_index("s") * 2 + lax.axis_index("c")
    base = wid * _BPW
    pltpu.sync_copy(idx_hbm.at[pl.ds(base, _BPW)], idx_v)
    for c in range(_BPW // _GC):
        pltpu.async_copy(
            table_hbm.at[idx_v.at[pl.ds(c * _GC, _GC)]], rows_v, sem).wait()
        pltpu.sync_copy(rows_v, out_hbm.at[pl.ds(base + c * _GC, _GC)])


@jax.jit
def _gather_call(embed, idx):
    f = functools.partial(
        pl.kernel,
        out_type=jax.ShapeDtypeStruct((N_TOK, EMBED_DIM), jnp.float32),
        mesh=plsc.VectorSubcoreMesh(core_axis_name="c", subcore_axis_name="s"),
        scratch_types=[
            pltpu.VMEM((_BPW,), jnp.int32),
            pltpu.VMEM((_GC, EMBED_DIM), jnp.float32),
            pltpu.SemaphoreType.DMA,
        ],
    )(_gather_body)
    return f(embed, idx)


def kernel(inputs, W_body, b_body, embed):
    idx3 = _argmin_call(inputs, W_body, b_body.reshape(1, EMBED_DIM), embed)
    idx = idx3.reshape(N_TOK)
    return _gather_call(embed, idx)


# speedup vs baseline: 1.7071x; 1.1968x over previous
import functools

import jax
import jax.numpy as jnp
from jax import lax
from jax.experimental import pallas as pl
from jax.experimental.pallas import tpu as pltpu
from jax.experimental.pallas import tpu_sc as plsc

N_TOK = 16384
IN_DIM = 768
EMBED_DIM = 256
N_EMBED = 8192

TM = 2048
MT = N_TOK // TM
NC = 2048
NCHUNK = N_EMBED // NC

_BIG = 3.0e38


def _argmin_body(x_ref, w_ref, b_ref, e_ref, idx_ref, esq_ref):
    @pl.when(pl.program_id(0) == 0)
    def _():
        e = e_ref[...]
        esq_ref[0, :] = jnp.sum(e * e, axis=1)

    xs = lax.dot_general(x_ref[...], w_ref[...], (((1,), (0,)), ((), ())),
                         preferred_element_type=jnp.float32)
    xs = xs + b_ref[...]
    xsq = jnp.sum(xs * xs, axis=1, keepdims=True)

    def chunk(c, carry):
        best_val, best_idx = carry
        e_c = e_ref[pl.ds(c * NC, NC), :]
        mm = lax.dot_general(xs, e_c, (((1,), (1,)), ((), ())),
                             preferred_element_type=jnp.float32)
        esq_c = esq_ref[:, pl.ds(c * NC, NC)]
        dist = (xsq + esq_c) - 2.0 * mm
        m = jnp.min(dist, axis=1, keepdims=True)
        lane = lax.broadcasted_iota(jnp.int32, (TM, NC), 1).astype(jnp.float32)
        cand = jnp.where(dist == m, lane, _BIG)
        a = jnp.min(cand, axis=1, keepdims=True) + jnp.float32(NC) * c
        upd = m < best_val
        return jnp.where(upd, m, best_val), jnp.where(upd, a, best_idx)

    init = (jnp.full((TM, 1), _BIG, jnp.float32),
            jnp.zeros((TM, 1), jnp.float32))
    _, best_idx = lax.fori_loop(0, NCHUNK, chunk, init)
    idx_ref[...] = best_idx.astype(jnp.int32).reshape(1, 1, TM)


@jax.jit
def _argmin_call(inputs, w, b2d, embed):
    return pl.pallas_call(
        _argmin_body,
        grid=(MT,),
        in_specs=[
            pl.BlockSpec((TM, IN_DIM), lambda i: (i, 0)),
            pl.BlockSpec((IN_DIM, EMBED_DIM), lambda i: (0, 0)),
            pl.BlockSpec((1, EMBED_DIM), lambda i: (0, 0)),
            pl.BlockSpec((N_EMBED, EMBED_DIM), lambda i: (0, 0)),
        ],
        out_specs=pl.BlockSpec((1, 1, TM), lambda i: (i, 0, 0)),
        out_shape=jax.ShapeDtypeStruct((MT, 1, TM), jnp.int32),
        scratch_shapes=[pltpu.VMEM((1, N_EMBED), jnp.float32)],
        compiler_params=pltpu.CompilerParams(
            dimension_semantics=("arbitrary",)),
    )(inputs, w, b2d, embed)


_NW = 32
_BPW = N_TOK // _NW
_GC = 256


def _gather_body(table_hbm, idx_hbm, out_hbm, idx_v, rows_v, sem):
    wid = lax.axis_index("s") * 2 + lax.axis_index("c")
    base = wid * _BPW
    pltpu.sync_copy(idx_hbm.at[pl.ds(base, _BPW)], idx_v)
    for c in range(_BPW // _GC):
        pltpu.async_copy(
            table_hbm.at[idx_v.at[pl.ds(c * _GC, _GC)]], rows_v, sem).wait()
        pltpu.sync_copy(rows_v, out_hbm.at[pl.ds(base + c * _GC, _GC)])


@jax.jit
def _gather_call(embed, idx):
    f = functools.partial(
        pl.kernel,
        out_type=jax.ShapeDtypeStruct((N_TOK, EMBED_DIM), jnp.float32),
        mesh=plsc.VectorSubcoreMesh(core_axis_name="c", subcore_axis_name="s"),
        scratch_types=[
            pltpu.VMEM((_BPW,), jnp.int32),
            pltpu.VMEM((_GC, EMBED_DIM), jnp.float32),
            pltpu.SemaphoreType.DMA,
        ],
    )(_gather_body)
    return f(embed, idx)


def kernel(inputs, W_body, b_body, embed):
    idx3 = _argmin_call(inputs, W_body, b_body.reshape(1, EMBED_DIM), embed)
    idx = idx3.reshape(N_TOK)
    return _gather_call(embed, idx)


# speedup vs baseline: 1.8426x; 1.0794x over previous
import functools

import jax
import jax.numpy as jnp
from jax import lax
from jax.experimental import pallas as pl
from jax.experimental.pallas import tpu as pltpu
from jax.experimental.pallas import tpu_sc as plsc

N_TOK = 16384
IN_DIM = 768
EMBED_DIM = 256
N_EMBED = 8192

TM = 2048
MT = N_TOK // TM
NC = 4096
NCHUNK = N_EMBED // NC

_BIG = 3.0e38


def _argmin_body(x_ref, w_ref, b_ref, e_ref, idx_ref, esq_ref):
    @pl.when(pl.program_id(0) == 0)
    def _():
        e = e_ref[...]
        esq_ref[0, :] = jnp.sum(e * e, axis=1)

    xs = lax.dot_general(x_ref[...], w_ref[...], (((1,), (0,)), ((), ())),
                         preferred_element_type=jnp.float32)
    xs = xs + b_ref[...]
    xsq = jnp.sum(xs * xs, axis=1, keepdims=True)

    def chunk(c, carry):
        best_val, best_idx = carry
        e_c = e_ref[pl.ds(c * NC, NC), :]
        mm = lax.dot_general(xs, e_c, (((1,), (1,)), ((), ())),
                             preferred_element_type=jnp.float32)
        esq_c = esq_ref[:, pl.ds(c * NC, NC)]
        dist = (xsq + esq_c) - 2.0 * mm
        m = jnp.min(dist, axis=1, keepdims=True)
        lane = lax.broadcasted_iota(jnp.int32, (TM, NC), 1).astype(jnp.float32)
        cand = jnp.where(dist == m, lane, _BIG)
        a = jnp.min(cand, axis=1, keepdims=True) + jnp.float32(NC) * c
        upd = m < best_val
        return jnp.where(upd, m, best_val), jnp.where(upd, a, best_idx)

    init = (jnp.full((TM, 1), _BIG, jnp.float32),
            jnp.zeros((TM, 1), jnp.float32))
    _, best_idx = lax.fori_loop(0, NCHUNK, chunk, init)
    idx_ref[...] = best_idx.astype(jnp.int32).reshape(1, 1, TM)


@jax.jit
def _argmin_call(inputs, w, b2d, embed):
    return pl.pallas_call(
        _argmin_body,
        grid=(MT,),
        in_specs=[
            pl.BlockSpec((TM, IN_DIM), lambda i: (i, 0)),
            pl.BlockSpec((IN_DIM, EMBED_DIM), lambda i: (0, 0)),
            pl.BlockSpec((1, EMBED_DIM), lambda i: (0, 0)),
            pl.BlockSpec((N_EMBED, EMBED_DIM), lambda i: (0, 0)),
        ],
        out_specs=pl.BlockSpec((1, 1, TM), lambda i: (i, 0, 0)),
        out_shape=jax.ShapeDtypeStruct((MT, 1, TM), jnp.int32),
        scratch_shapes=[pltpu.VMEM((1, N_EMBED), jnp.float32)],
        compiler_params=pltpu.CompilerParams(
            dimension_semantics=("arbitrary",)),
    )(inputs, w, b2d, embed)


_NW = 32
_BPW = N_TOK // _NW
_GC = 256


def _gather_body(table_hbm, idx_hbm, out_hbm, idx_v, rows_v, sem):
    wid = lax.axis_index("s") * 2 + lax.axis_index("c")
    base = wid * _BPW
    pltpu.sync_copy(idx_hbm.at[pl.ds(base, _BPW)], idx_v)
    for c in range(_BPW // _GC):
        pltpu.async_copy(
            table_hbm.at[idx_v.at[pl.ds(c * _GC, _GC)]], rows_v, sem).wait()
        pltpu.sync_copy(rows_v, out_hbm.at[pl.ds(base + c * _GC, _GC)])


@jax.jit
def _gather_call(embed, idx):
    f = functools.partial(
        pl.kernel,
        out_type=jax.ShapeDtypeStruct((N_TOK, EMBED_DIM), jnp.float32),
        mesh=plsc.VectorSubcoreMesh(core_axis_name="c", subcore_axis_name="s"),
        scratch_types=[
            pltpu.VMEM((_BPW,), jnp.int32),
            pltpu.VMEM((_GC, EMBED_DIM), jnp.float32),
            pltpu.SemaphoreType.DMA,
        ],
    )(_gather_body)
    return f(embed, idx)


def kernel(inputs, W_body, b_body, embed):
    idx3 = _argmin_call(inputs, W_body, b_body.reshape(1, EMBED_DIM), embed)
    idx = idx3.reshape(N_TOK)
    return _gather_call(embed, idx)
